# baseline (speedup 1.0000x reference)
import jax
import jax.numpy as jnp
from jax import lax
from jax.experimental import pallas as pl
from jax.experimental.pallas import tpu as pltpu

_D = 2048
_E = 16
_TOPK = 2
_T = 16384
_BT = 1024
_DC = _D // 2


def _router_body(h_ref, wt_ref, b_ref, logits_ref, wts_ref, sel_ref, mask_ref):
    d = pl.program_id(1)
    partial = jnp.dot(h_ref[...], wt_ref[...],
                      preferred_element_type=jnp.float32).T

    @pl.when(d == 0)
    def _():
        logits_ref[...] = partial + b_ref[...]

    @pl.when(d == 1)
    def _():
        lt = logits_ref[...] + partial
        logits_ref[...] = lt

        e_iota = lax.broadcasted_iota(jnp.int32, (_E, _BT), 0)
        v1 = jnp.max(lt, axis=0, keepdims=True)
        i1 = jnp.min(jnp.where(lt == v1, e_iota, _E), axis=0, keepdims=True)
        l2 = jnp.where(e_iota == i1, jnp.float32(-jnp.inf), lt)
        v2 = jnp.max(l2, axis=0, keepdims=True)
        i2 = jnp.min(jnp.where(l2 == v2, e_iota, _E), axis=0, keepdims=True)

        e2 = jnp.exp(v2 - v1)
        denom = 1.0 + e2
        wts_ref[...] = jnp.concatenate([1.0 / denom, e2 / denom], axis=0)
        sel_ref[...] = jnp.concatenate([i1, i2], axis=0)

        r_iota = lax.broadcasted_iota(jnp.int32, (_E * _TOPK, _BT), 0)
        sel_r = jnp.where((r_iota & 1) == 0, i1, i2)
        mask_ref[...] = (sel_r == (r_iota >> 1)).astype(jnp.int32)


def kernel(hidden_states, W, b):
    wt = W.T
    b2 = b.reshape(_E, 1)
    grid = (_T // _BT, 2)
    logits_t, wts_t, sel_t, mask_t = pl.pallas_call(
        _router_body,
        grid=grid,
        in_specs=[
            pl.BlockSpec((_BT, _DC), lambda i, d: (i, d)),
            pl.BlockSpec((_DC, _E), lambda i, d: (d, 0)),
            pl.BlockSpec((_E, 1), lambda i, d: (0, 0)),
        ],
        out_specs=[
            pl.BlockSpec((_E, _BT), lambda i, d: (0, i)),
            pl.BlockSpec((_TOPK, _BT), lambda i, d: (0, i)),
            pl.BlockSpec((_TOPK, _BT), lambda i, d: (0, i)),
            pl.BlockSpec((_E * _TOPK, _BT), lambda i, d: (0, i)),
        ],
        out_shape=[
            jax.ShapeDtypeStruct((_E, _T), jnp.float32),
            jax.ShapeDtypeStruct((_TOPK, _T), jnp.float32),
            jax.ShapeDtypeStruct((_TOPK, _T), jnp.int32),
            jax.ShapeDtypeStruct((_E * _TOPK, _T), jnp.int32),
        ],
        compiler_params=pltpu.CompilerParams(
            dimension_semantics=("parallel", "arbitrary"),
        ),
    )(hidden_states, wt, b2)
    return (logits_t.T, wts_t.T, sel_t.T, mask_t.reshape(_E, _TOPK, _T))


# speedup vs baseline: 1.2198x; 1.2198x over previous
import jax
import jax.numpy as jnp
from jax import lax
from jax.experimental import pallas as pl
from jax.experimental.pallas import tpu as pltpu

_D = 2048
_E = 16
_TOPK = 2
_T = 16384
_CH = 512
_NS = _T // _CH
_NB = 4


def _router_body(h_hbm, wt_ref, b_ref, logits_ref, wts_ref, sel_ref, mask_ref,
                 hbuf, sems):
    wt = wt_ref[...]
    b_col = b_ref[...]

    def copy_in(step, buf):
        return pltpu.make_async_copy(
            h_hbm.at[pl.ds(step * _CH, _CH), :], hbuf.at[buf], sems.at[buf])

    for s in range(_NB):
        copy_in(s, s).start()

    def step_fn(s, carry):
        buf = lax.rem(s, _NB)
        copy_in(s, buf).wait()
        h = hbuf[buf]
        lt = jnp.dot(h, wt, preferred_element_type=jnp.float32).T + b_col
        logits_ref[:, pl.ds(s * _CH, _CH)] = lt

        e_iota = lax.broadcasted_iota(jnp.int32, (_E, _CH), 0)
        v1 = jnp.max(lt, axis=0, keepdims=True)
        i1 = jnp.min(jnp.where(lt == v1, e_iota, _E), axis=0, keepdims=True)
        l2 = jnp.where(e_iota == i1, jnp.float32(-jnp.inf), lt)
        v2 = jnp.max(l2, axis=0, keepdims=True)
        i2 = jnp.min(jnp.where(l2 == v2, e_iota, _E), axis=0, keepdims=True)

        e2 = jnp.exp(v2 - v1)
        denom = 1.0 + e2
        wts_ref[:, pl.ds(s * _CH, _CH)] = jnp.concatenate(
            [1.0 / denom, e2 / denom], axis=0)
        sel_ref[:, pl.ds(s * _CH, _CH)] = jnp.concatenate([i1, i2], axis=0)

        r_iota = lax.broadcasted_iota(jnp.int32, (_E * _TOPK, _CH), 0)
        sel_r = jnp.where((r_iota & 1) == 0, i1, i2)
        mask_ref[:, pl.ds(s * _CH, _CH)] = (sel_r == (r_iota >> 1)).astype(
            jnp.int32)

        nxt = s + _NB

        @pl.when(nxt < _NS)
        def _():
            copy_in(nxt, buf).start()

        return carry

    lax.fori_loop(0, _NS, step_fn, 0)


def kernel(hidden_states, W, b):
    wt = W.T
    b2 = b.reshape(_E, 1)
    logits_t, wts_t, sel_t, mask_t = pl.pallas_call(
        _router_body,
        in_specs=[
            pl.BlockSpec(memory_space=pl.ANY),
            pl.BlockSpec((_D, _E), lambda: (0, 0)),
            pl.BlockSpec((_E, 1), lambda: (0, 0)),
        ],
        out_specs=[
            pl.BlockSpec((_E, _T), lambda: (0, 0)),
            pl.BlockSpec((_TOPK, _T), lambda: (0, 0)),
            pl.BlockSpec((_TOPK, _T), lambda: (0, 0)),
            pl.BlockSpec((_E * _TOPK, _T), lambda: (0, 0)),
        ],
        out_shape=[
            jax.ShapeDtypeStruct((_E, _T), jnp.float32),
            jax.ShapeDtypeStruct((_TOPK, _T), jnp.float32),
            jax.ShapeDtypeStruct((_TOPK, _T), jnp.int32),
            jax.ShapeDtypeStruct((_E * _TOPK, _T), jnp.int32),
        ],
        scratch_shapes=[
            pltpu.VMEM((_NB, _CH, _D), jnp.float32),
            pltpu.SemaphoreType.DMA((_NB,)),
        ],
    )(hidden_states, wt, b2)
    return (logits_t.T, wts_t.T, sel_t.T, mask_t.reshape(_E, _TOPK, _T))
